# baseline (speedup 1.0000x reference)
import functools

import jax
import jax.numpy as jnp
from jax import lax
from jax.experimental import pallas as pl
from jax.experimental.pallas import tpu as pltpu
from jax.experimental.pallas import tpu_sc as plsc

ALPHA = 0.2
N_ROWS = 16384
N_COLS = 768
B = 16
NC = 2
NS = 16
NW = NC * NS
CHUNK = 32
N_CHUNKS = N_ROWS // CHUNK // NW
LANES = 16
VPR = N_COLS // LANES


def _sc_mix(x, bo_mat, be_mat, bd_mat, lam_vec):
    mesh = plsc.VectorSubcoreMesh(core_axis_name="c", subcore_axis_name="s")

    @functools.partial(
        pl.kernel,
        out_type=jax.ShapeDtypeStruct((N_ROWS, N_COLS), jnp.float32),
        mesh=mesh,
        compiler_params=pltpu.CompilerParams(needs_layout_passes=False),
        scratch_types=[
            pltpu.VMEM((B, LANES), jnp.int32),
            pltpu.VMEM((B, LANES), jnp.int32),
            pltpu.VMEM((B, LANES), jnp.int32),
            pltpu.VMEM((LANES,), jnp.float32),
            pltpu.VMEM((CHUNK, N_COLS), jnp.float32),
            pltpu.VMEM((CHUNK, N_COLS), jnp.float32),
            pltpu.VMEM((CHUNK, N_COLS), jnp.float32),
            pltpu.VMEM((CHUNK, N_COLS), jnp.float32),
            pltpu.VMEM((CHUNK, N_COLS), jnp.float32),
            pltpu.SemaphoreType.DMA,
            pltpu.SemaphoreType.DMA,
            pltpu.SemaphoreType.DMA,
            pltpu.SemaphoreType.DMA,
            pltpu.SemaphoreType.DMA,
            pltpu.SemaphoreType.DMA,
            pltpu.SemaphoreType.DMA,
            pltpu.SemaphoreType.DMA,
        ],
    )
    def kfn(x_hbm, bo_hbm, be_hbm, bd_hbm, lam_hbm, out_hbm,
            bo_v, be_v, bd_v, lam_v, xbuf0, xbuf1, xbuf2, obuf0, obuf1,
            sem_a0, sem_a1, sem_a2, sem_b0, sem_b1,
            sem_c0, sem_c1, sem_c2):
        cid = lax.axis_index("c")
        sid = lax.axis_index("s")
        wid = sid * NC + cid

        pltpu.sync_copy(bo_hbm, bo_v)
        pltpu.sync_copy(be_hbm, be_v)
        pltpu.sync_copy(bd_hbm, bd_v)
        pltpu.sync_copy(lam_hbm, lam_v)

        bo = [bo_v[k, :] for k in range(B)]
        be = [be_v[k, :] for k in range(B)]
        bd = [bd_v[k, :] for k in range(B)]
        lam_r = lam_v[...]

        xbufs = (xbuf0, xbuf1, xbuf2)
        obufs = (obuf0, obuf1)
        asems = (sem_a0, sem_a1, sem_a2)
        bsems = (sem_b0, sem_b1)
        csems = (sem_c0, sem_c1, sem_c2)

        def base_of(t):
            return (wid + t * NW) * CHUNK

        def issue_load(t):
            base = base_of(t)
            srcs = []
            mixed_any = None
            for v in range(CHUNK // LANES):
                rv = base + v * LANES + lax.iota(jnp.int32, LANES)
                src = rv
                for k in range(B):
                    msk = (rv >= bo[k]) & (rv < be[k])
                    src = jnp.where(msk, rv + bd[k], src)
                srcs.append(src)
                m = jnp.any(src != rv)
                mixed_any = m if mixed_any is None else (mixed_any | m)
            pltpu.async_copy(x_hbm.at[pl.ds(base, CHUNK)], xbufs[t % 3],
                             asems[t % 3])
            return base, srcs, mixed_any

        def issue_gathers(t):
            _, srcs, mixed_any = infos[t]

            @pl.when(mixed_any)
            def _start_gathers():
                for v in range(CHUNK // LANES):
                    pltpu.async_copy(
                        x_hbm.at[srcs[v]],
                        obufs[t % 2].at[pl.ds(v * LANES, LANES)],
                        bsems[t % 2])

        infos = {0: issue_load(0)}
        if N_CHUNKS > 1:
            infos[1] = issue_load(1)
        issue_gathers(0)
        for t in range(N_CHUNKS):
            if t + 2 < N_CHUNKS:
                if t >= 1:
                    pb = base_of(t - 1)
                    pltpu.make_async_copy(
                        xbufs[(t + 2) % 3],
                        out_hbm.at[pl.ds(pb, CHUNK)],
                        csems[(t + 2) % 3]).wait()
                infos[t + 2] = issue_load(t + 2)
            if t + 1 < N_CHUNKS:
                issue_gathers(t + 1)
            base, srcs, mixed_any = infos.pop(t)
            pltpu.make_async_copy(x_hbm.at[pl.ds(base, CHUNK)],
                                  xbufs[t % 3], asems[t % 3]).wait()

            @pl.when(mixed_any)
            def _blend():
                for v in range(CHUNK // LANES):
                    pltpu.make_async_copy(
                        x_hbm.at[srcs[v]],
                        obufs[t % 2].at[pl.ds(v * LANES, LANES)],
                        bsems[t % 2]).wait()

                def row_body(r, carry):
                    for d in range(VPR):
                        sl = pl.ds(d * LANES, LANES)
                        xs = xbufs[t % 3][r, sl]
                        ot = obufs[t % 2][r, sl]
                        xbufs[t % 3][r, sl] = ot + lam_r * (xs - ot)
                    return carry

                lax.fori_loop(0, CHUNK, row_body, 0)

            pltpu.async_copy(xbufs[t % 3], out_hbm.at[pl.ds(base, CHUNK)],
                             csems[t % 3])

        for t in (N_CHUNKS - 3, N_CHUNKS - 2, N_CHUNKS - 1):
            pltpu.make_async_copy(
                xbufs[t % 3],
                out_hbm.at[pl.ds(base_of(t), CHUNK)],
                csems[t % 3]).wait()

    return kfn(x, bo_mat, be_mat, bd_mat, lam_vec)


def kernel(patch_embs, n_patches_list):
    key = jax.random.key(42)
    ka, kb = jax.random.split(key)
    lam = jax.random.beta(ka, ALPHA, ALPHA)
    lam = jnp.maximum(lam, 1.0 - lam)
    perm = jax.random.permutation(kb, B).astype(jnp.int32)

    sizes = n_patches_list.astype(jnp.int32)
    offs = jnp.concatenate(
        [jnp.zeros((1,), jnp.int32), jnp.cumsum(sizes)[:-1]])
    n_mix = jnp.minimum(sizes, sizes[perm])
    ends = offs + n_mix
    dlt = offs[perm] - offs
    bo_mat = jnp.broadcast_to(offs[:, None], (B, LANES))
    be_mat = jnp.broadcast_to(ends[:, None], (B, LANES))
    bd_mat = jnp.broadcast_to(dlt[:, None], (B, LANES))
    lam_vec = jnp.full((LANES,), lam, dtype=jnp.float32)

    mixed = _sc_mix(patch_embs, bo_mat, be_mat, bd_mat, lam_vec)
    return (mixed, jnp.asarray(lam, dtype=jnp.float32), perm)


# speedup vs baseline: 1.0005x; 1.0005x over previous
import functools

import jax
import jax.numpy as jnp
from jax import lax
from jax.experimental import pallas as pl
from jax.experimental.pallas import tpu as pltpu
from jax.experimental.pallas import tpu_sc as plsc

ALPHA = 0.2
N_ROWS = 16384
N_COLS = 768
B = 16
NC = 2
NS = 16
NW = NC * NS
CHUNK = 32
N_CHUNKS = N_ROWS // CHUNK // NW
LANES = 16
VPR = N_COLS // LANES


def _sc_mix(x, bo_mat, be_mat, bd_mat, lam_vec):
    mesh = plsc.VectorSubcoreMesh(core_axis_name="c", subcore_axis_name="s")

    @functools.partial(
        pl.kernel,
        out_type=jax.ShapeDtypeStruct((N_ROWS, N_COLS), jnp.float32),
        mesh=mesh,
        compiler_params=pltpu.CompilerParams(needs_layout_passes=False),
        scratch_types=[
            pltpu.VMEM((B, LANES), jnp.int32),
            pltpu.VMEM((B, LANES), jnp.int32),
            pltpu.VMEM((B, LANES), jnp.int32),
            pltpu.VMEM((LANES,), jnp.float32),
            pltpu.VMEM((CHUNK, N_COLS), jnp.float32),
            pltpu.VMEM((CHUNK, N_COLS), jnp.float32),
            pltpu.VMEM((CHUNK, N_COLS), jnp.float32),
            pltpu.VMEM((CHUNK, N_COLS), jnp.float32),
            pltpu.VMEM((CHUNK, N_COLS), jnp.float32),
            pltpu.SemaphoreType.DMA,
            pltpu.SemaphoreType.DMA,
            pltpu.SemaphoreType.DMA,
            pltpu.SemaphoreType.DMA,
            pltpu.SemaphoreType.DMA,
            pltpu.SemaphoreType.DMA,
            pltpu.SemaphoreType.DMA,
            pltpu.SemaphoreType.DMA,
        ],
    )
    def kfn(x_hbm, bo_hbm, be_hbm, bd_hbm, lam_hbm, out_hbm,
            bo_v, be_v, bd_v, lam_v, xbuf0, xbuf1, xbuf2, obuf0, obuf1,
            sem_a0, sem_a1, sem_a2, sem_b0, sem_b1,
            sem_c0, sem_c1, sem_c2):
        cid = lax.axis_index("c")
        sid = lax.axis_index("s")
        wid = sid * NC + cid

        pltpu.sync_copy(bo_hbm, bo_v)
        pltpu.sync_copy(be_hbm, be_v)
        pltpu.sync_copy(bd_hbm, bd_v)
        pltpu.sync_copy(lam_hbm, lam_v)

        bo = [bo_v[k, :] for k in range(B)]
        be = [be_v[k, :] for k in range(B)]
        bd = [bd_v[k, :] for k in range(B)]
        lam_r = lam_v[...]

        xbufs = (xbuf0, xbuf1, xbuf2)
        obufs = (obuf0, obuf1)
        asems = (sem_a0, sem_a1, sem_a2)
        bsems = (sem_b0, sem_b1)
        csems = (sem_c0, sem_c1, sem_c2)

        def base_of(t):
            return (wid + t * NW) * CHUNK

        def issue_load(t):
            base = base_of(t)
            srcs = []
            mixs = []
            for v in range(CHUNK // LANES):
                rv = base + v * LANES + lax.iota(jnp.int32, LANES)
                src = rv
                for k in range(B):
                    msk = (rv >= bo[k]) & (rv < be[k])
                    src = jnp.where(msk, rv + bd[k], src)
                srcs.append(src)
                mixs.append(jnp.any(src != rv))
            mixed_any = mixs[0]
            for m in mixs[1:]:
                mixed_any = mixed_any | m
            pltpu.async_copy(x_hbm.at[pl.ds(base, CHUNK)], xbufs[t % 3],
                             asems[t % 3])
            return base, srcs, mixed_any

        def issue_gathers(t):
            _, srcs, mixed_any = infos[t]

            @pl.when(mixed_any)
            def _start_gathers():
                for v in range(CHUNK // LANES):
                    pltpu.async_copy(
                        x_hbm.at[srcs[v]],
                        obufs[t % 2].at[pl.ds(v * LANES, LANES)],
                        bsems[t % 2])

        infos = {0: issue_load(0)}
        if N_CHUNKS > 1:
            infos[1] = issue_load(1)
        issue_gathers(0)
        for t in range(N_CHUNKS):
            if t + 2 < N_CHUNKS:
                if t >= 1:
                    pb = base_of(t - 1)
                    pltpu.make_async_copy(
                        xbufs[(t + 2) % 3],
                        out_hbm.at[pl.ds(pb, CHUNK)],
                        csems[(t + 2) % 3]).wait()
                infos[t + 2] = issue_load(t + 2)
            if t + 1 < N_CHUNKS:
                issue_gathers(t + 1)
            base, srcs, mixed_any = infos.pop(t)
            pltpu.make_async_copy(x_hbm.at[pl.ds(base, CHUNK)],
                                  xbufs[t % 3], asems[t % 3]).wait()

            @pl.when(mixed_any)
            def _blend():
                for v in range(CHUNK // LANES):
                    pltpu.make_async_copy(
                        x_hbm.at[srcs[v]],
                        obufs[t % 2].at[pl.ds(v * LANES, LANES)],
                        bsems[t % 2]).wait()

                def row_body(r, carry):
                    for d in range(VPR):
                        sl = pl.ds(d * LANES, LANES)
                        xs = xbufs[t % 3][r, sl]
                        ot = obufs[t % 2][r, sl]
                        xbufs[t % 3][r, sl] = ot + lam_r * (xs - ot)
                    return carry

                lax.fori_loop(0, CHUNK, row_body, 0)

            pltpu.async_copy(xbufs[t % 3], out_hbm.at[pl.ds(base, CHUNK)],
                             csems[t % 3])

        for t in (N_CHUNKS - 3, N_CHUNKS - 2, N_CHUNKS - 1):
            pltpu.make_async_copy(
                xbufs[t % 3],
                out_hbm.at[pl.ds(base_of(t), CHUNK)],
                csems[t % 3]).wait()

    return kfn(x, bo_mat, be_mat, bd_mat, lam_vec)


def kernel(patch_embs, n_patches_list):
    key = jax.random.key(42)
    ka, kb = jax.random.split(key)
    lam = jax.random.beta(ka, ALPHA, ALPHA)
    lam = jnp.maximum(lam, 1.0 - lam)
    perm = jax.random.permutation(kb, B).astype(jnp.int32)

    sizes = n_patches_list.astype(jnp.int32)
    offs = jnp.concatenate(
        [jnp.zeros((1,), jnp.int32), jnp.cumsum(sizes)[:-1]])
    n_mix = jnp.minimum(sizes, sizes[perm])
    ends = offs + n_mix
    dlt = offs[perm] - offs
    bo_mat = jnp.broadcast_to(offs[:, None], (B, LANES))
    be_mat = jnp.broadcast_to(ends[:, None], (B, LANES))
    bd_mat = jnp.broadcast_to(dlt[:, None], (B, LANES))
    lam_vec = jnp.full((LANES,), lam, dtype=jnp.float32)

    mixed = _sc_mix(patch_embs, bo_mat, be_mat, bd_mat, lam_vec)
    return (mixed, jnp.asarray(lam, dtype=jnp.float32), perm)
